# baseline (speedup 1.0000x reference)
import functools

import jax
import jax.numpy as jnp
from jax import lax
from jax.experimental import pallas as pl
from jax.experimental.pallas import tpu as pltpu
from jax.experimental.pallas import tpu_sc as plsc

D = 128


def _mm_kernel(x_ref, w_ref, b_ref, o_ref):
    o_ref[...] = (jnp.dot(x_ref[...], w_ref[...], preferred_element_type=jnp.float32)
                  + b_ref[...])


def _matmul_bias(x, w, b, block=1024):
    n, d = x.shape
    k = w.shape[1]
    grid = (n + block - 1) // block
    return pl.pallas_call(
        _mm_kernel,
        grid=(grid,),
        in_specs=[
            pl.BlockSpec((block, d), lambda i: (i, 0)),
            pl.BlockSpec((d, k), lambda i: (0, 0)),
            pl.BlockSpec((1, k), lambda i: (0, 0)),
        ],
        out_specs=pl.BlockSpec((block, k), lambda i: (i, 0)),
        out_shape=jax.ShapeDtypeStruct((n, k), jnp.float32),
    )(x, w, b.reshape(1, k))


def _sc_gather_scatter_small(table, src2d, dst2d, n_out):
    nb_tile = src2d.shape[0] // 32
    gb = src2d.shape[1]
    n_pad = ((n_out + 127) // 128) * 128
    rows_tile = n_pad // 16

    mesh = plsc.VectorSubcoreMesh(core_axis_name="c", subcore_axis_name="s")

    @functools.partial(
        pl.kernel,
        out_type=jax.ShapeDtypeStruct((2, n_pad, D), jnp.float32),
        mesh=mesh,
        compiler_params=pltpu.CompilerParams(needs_layout_passes=False),
        scratch_types=[
            pltpu.VMEM_SHARED((n_pad, D), jnp.float32),
            pltpu.VMEM((nb_tile, gb), jnp.int32),
            pltpu.VMEM((nb_tile, gb), jnp.int32),
            pltpu.VMEM((gb, D), jnp.float32),
        ],
    )
    def k(table_hbm, src_hbm, dst_hbm, out_hbm, acc, sbuf, dbuf, rowbuf):
        c = lax.axis_index("c")
        s = lax.axis_index("s")
        wid = c * 16 + s
        pltpu.sync_copy(src_hbm.at[pl.ds(wid * nb_tile, nb_tile)], sbuf)
        pltpu.sync_copy(dst_hbm.at[pl.ds(wid * nb_tile, nb_tile)], dbuf)

        def _zrow(r, _):
            for l in range(D // 16):
                rowbuf[r, pl.ds(l * 16, 16)] = jnp.zeros((16,), jnp.float32)
            return 0
        lax.fori_loop(0, gb, _zrow, 0)
        for z in range(rows_tile // gb):
            pltpu.sync_copy(rowbuf, acc.at[pl.ds(s * rows_tile + z * gb, gb)])
        rem = rows_tile % gb
        if rem:
            pltpu.sync_copy(rowbuf.at[pl.ds(0, rem)],
                            acc.at[pl.ds(s * rows_tile + (rows_tile // gb) * gb, rem)])
        plsc.subcore_barrier()

        def body(j, _):
            pltpu.sync_copy(table_hbm.at[sbuf.at[j]], rowbuf)
            def _relu(r, _):
                for l in range(D // 16):
                    v = rowbuf[r, pl.ds(l * 16, 16)]
                    rowbuf[r, pl.ds(l * 16, 16)] = jnp.maximum(v, 0.0)
                return 0
            lax.fori_loop(0, gb, _relu, 0)
            pltpu.sync_copy(rowbuf, acc.at[dbuf.at[j]], add=True)
            return 0
        lax.fori_loop(0, nb_tile, body, 0)
        plsc.subcore_barrier()
        pltpu.sync_copy(acc.at[pl.ds(s * rows_tile, rows_tile)],
                        out_hbm.at[c, pl.ds(s * rows_tile, rows_tile)])

    part = k(table, src2d, dst2d)
    return part[0, :n_out] + part[1, :n_out]


_R_CHUNK = 8960
_TB = 2000
_CAPL = 256


def _sc_scatter_chunked(tables, fams, n_out):
    r = _R_CHUNK
    n_chunks = (n_out + r - 1) // r
    if n_chunks % 2:
        n_chunks += 1
    n_pad = n_chunks * r
    capl = _CAPL

    mesh = plsc.VectorSubcoreMesh(core_axis_name="c", subcore_axis_name="s")

    @functools.partial(
        pl.kernel,
        out_type=jax.ShapeDtypeStruct((n_pad, D), jnp.float32),
        mesh=mesh,
        compiler_params=pltpu.CompilerParams(needs_layout_passes=False),
        scratch_types=[
            pltpu.VMEM_SHARED((r + 16, D), jnp.float32),
            pltpu.VMEM((_TB,), jnp.int32),
            pltpu.VMEM((_TB,), jnp.int32),
            pltpu.VMEM((_TB,), jnp.int32),
            pltpu.VMEM((16 * _CAPL,), jnp.int32),
            pltpu.VMEM((16 * _CAPL,), jnp.int32),
            pltpu.VMEM((16 * _CAPL,), jnp.int32),
            pltpu.VMEM((128, D), jnp.float32),
            pltpu.VMEM((128, D), jnp.float32),
            pltpu.VMEM((128,), jnp.int32),
            pltpu.SemaphoreType.DMA,
            pltpu.SemaphoreType.DMA,
            pltpu.SemaphoreType.DMA,
        ],
    )
    def k(*refs):
        n_tbl = len(tables)
        tbl = refs[:n_tbl]
        idx = refs[n_tbl:n_tbl + 3 * len(fams)]
        out_hbm = refs[n_tbl + 3 * len(fams)]
        (acc, dbat, abat, bbat, dloc, sa, sb, rowa, rowb, dstage,
         sem0, sem1, sem2) = refs[n_tbl + 3 * len(fams) + 1:]
        c = lax.axis_index("c")
        s = lax.axis_index("s")
        rows_tile = r // 16

        def _scan_fam(fi, lo):
            ia, ib, idm = idx[3 * fi], idx[3 * fi + 1], idx[3 * fi + 2]
            t16 = ia.shape[0] // 16
            base = s * t16
            laneoff = lax.iota(jnp.int32, 16) * capl
            rsplat = jnp.full((16,), r, jnp.int32)

            def _fill(i, _):
                dloc[pl.ds(i * 16, 16)] = rsplat
                return 0
            lax.fori_loop(0, capl, _fill, 0)

            def process(cv):
                nb = (jnp.max(cv) + 127) // 128

                def lane_loop(ll, _):
                    def gloop(g, _):
                        o = ll * capl + g * 128
                        pltpu.sync_copy(tbl[fams[fi][0]].at[sa.at[pl.ds(o, 128)]], rowa)
                        pltpu.sync_copy(tbl[fams[fi][1]].at[sb.at[pl.ds(o, 128)]], rowb)

                        def rr(rw, _):
                            for l in range(D // 16):
                                v = rowa[rw, pl.ds(l * 16, 16)] + rowb[rw, pl.ds(l * 16, 16)]
                                rowa[rw, pl.ds(l * 16, 16)] = jnp.maximum(v, 0.0)
                            return 0
                        lax.fori_loop(0, 128, rr, 0)
                        for i in range(8):
                            dstage[pl.ds(i * 16, 16)] = dloc[pl.ds(o + i * 16, 16)]
                            dloc[pl.ds(o + i * 16, 16)] = rsplat
                        pltpu.sync_copy(rowa, acc.at[dstage], add=True)
                        return 0
                    lax.fori_loop(0, nb, gloop, 0)
                    return 0
                lax.fori_loop(0, 16, lane_loop, 0)
                return jnp.zeros((16,), jnp.int32)

            def batch_body(bi, cv):
                off = base + bi * _TB
                cpd = pltpu.async_copy(idm.at[pl.ds(off, _TB)], dbat, sem0)
                cpa = pltpu.async_copy(ia.at[pl.ds(off, _TB)], abat, sem1)
                cpb = pltpu.async_copy(ib.at[pl.ds(off, _TB)], bbat, sem2)
                cpd.wait()
                cpa.wait()
                cpb.wait()

                def grp(i, cv):
                    d = dbat[pl.ds(i * 16, 16)]
                    a = abat[pl.ds(i * 16, 16)]
                    b = bbat[pl.ds(i * 16, 16)]
                    m = (d >= lo) & (d < lo + r)
                    dv = jnp.where(m, d - lo, rsplat)
                    pos = laneoff + cv
                    plsc.store_scatter(dloc, [pos], dv)
                    plsc.store_scatter(sa, [pos], a)
                    plsc.store_scatter(sb, [pos], b)
                    return cv + m.astype(jnp.int32)
                cv = lax.fori_loop(0, _TB // 16, grp, cv)
                return lax.cond(jnp.max(cv) > capl - _TB // 16 - 1,
                                process, lambda x: x, cv)

            cv = lax.fori_loop(0, t16 // _TB, batch_body,
                               jnp.zeros((16,), jnp.int32))
            process(cv)

        def _zidx(i, _):
            z16 = jnp.zeros((16,), jnp.int32)
            sa[pl.ds(i * 16, 16)] = z16
            sb[pl.ds(i * 16, 16)] = z16
            return 0
        lax.fori_loop(0, capl, _zidx, 0)

        def chunk_body(z, _):
            lo = (2 * z + c) * r
            def _zrow(rw, _):
                for l in range(D // 16):
                    rowa[rw, pl.ds(l * 16, 16)] = jnp.zeros((16,), jnp.float32)
                return 0
            lax.fori_loop(0, 128, _zrow, 0)
            nfull = rows_tile // 128
            for q in range(nfull):
                pltpu.sync_copy(rowa, acc.at[pl.ds(s * rows_tile + q * 128, 128)])
            rem = rows_tile - nfull * 128
            if rem:
                pltpu.sync_copy(rowa.at[pl.ds(0, rem)],
                                acc.at[pl.ds(s * rows_tile + nfull * 128, rem)])
            plsc.subcore_barrier()
            for fi in range(len(fams)):
                _scan_fam(fi, lo)
            plsc.subcore_barrier()
            pltpu.sync_copy(acc.at[pl.ds(s * rows_tile, rows_tile)],
                            out_hbm.at[pl.ds(lo + s * rows_tile, rows_tile)])
            return 0
        lax.fori_loop(0, n_chunks // 2, chunk_body, 0)

    args = list(tables)
    for (_, _, ia, ib, idm) in fams:
        args += [ia, ib, idm]
    return k(*args)[:n_out]


def _sc_row_gather(table, idxv):
    n = idxv.shape[0]
    n_tile = n // 32
    nfull = n_tile // 128
    tail = n_tile - nfull * 128
    cap = n_tile + 128

    mesh = plsc.VectorSubcoreMesh(core_axis_name="c", subcore_axis_name="s")

    @functools.partial(
        pl.kernel,
        out_type=jax.ShapeDtypeStruct((n, D), jnp.float32),
        mesh=mesh,
        compiler_params=pltpu.CompilerParams(needs_layout_passes=False),
        scratch_types=[
            pltpu.VMEM((cap,), jnp.int32),
            pltpu.VMEM((128, D), jnp.float32),
        ],
    )
    def k(table_hbm, idx_hbm, out_hbm, ibuf, rowbuf):
        c = lax.axis_index("c")
        s = lax.axis_index("s")
        wid = c * 16 + s
        base = wid * n_tile
        pltpu.sync_copy(idx_hbm.at[pl.ds(base, n_tile)], ibuf.at[pl.ds(0, n_tile)])
        if tail:
            for i in range(8):
                ibuf[pl.ds(n_tile + i * 16, 16)] = jnp.zeros((16,), jnp.int32)

        def body(g, _):
            pltpu.sync_copy(table_hbm.at[ibuf.at[pl.ds(g * 128, 128)]], rowbuf)
            pltpu.sync_copy(rowbuf, out_hbm.at[pl.ds(base + g * 128, 128)])
            return 0
        lax.fori_loop(0, nfull, body, 0)
        if tail:
            pltpu.sync_copy(table_hbm.at[ibuf.at[pl.ds(nfull * 128, 128)]], rowbuf)
            pltpu.sync_copy(rowbuf.at[pl.ds(0, tail)],
                            out_hbm.at[pl.ds(base + nfull * 128, tail)])

    return k(table, idxv)


def _scatter(src, idx, size):
    return jnp.zeros((size, src.shape[1]), src.dtype).at[idx].add(src)


def kernel(edge_attr0, edge_attr1, edge_attr2, edge_index0, edge_index, edge_index2,
           triangle_0_1_1, triangle_1_1_1, triangle_1_1_2, triangle_1_2_2, triangle_2_2_2,
           inverse_edge_1, inverse_edge_2,
           proj0_W, proj0_b, proj1_W, proj1_b, proj2_W, proj2_b,
           mlp0_W1, mlp0_b1, mlp0_g, mlp0_beta, mlp0_W2, mlp0_b2,
           mlp1_W1, mlp1_b1, mlp1_g, mlp1_beta, mlp1_W2, mlp1_b2,
           mlp2_W1, mlp2_b1, mlp2_g, mlp2_beta, mlp2_W2, mlp2_b2,
           norm0_g, norm0_beta, norm1_g, norm1_beta, norm2_g, norm2_beta,
           eps0, eps1, eps2):
    e0, e1, e2 = edge_attr0, edge_attr1, edge_attr2
    num0, num1, num2 = e0.shape[0], e1.shape[0], e2.shape[0]

    w1cat = jnp.concatenate([2.0 * proj0_W, proj1_W, proj2_W], axis=1)
    b1cat = jnp.concatenate([proj0_b, 0.5 * proj1_b, 0.5 * proj2_b])
    p1cat = _matmul_bias(e1, w1cat, b1cat)
    A0, B1, B2 = p1cat[:, :D], p1cat[:, D:2 * D], p1cat[:, 2 * D:]
    w2cat = jnp.concatenate([proj1_W, proj2_W], axis=1)
    b2cat = jnp.concatenate([0.5 * proj1_b, 0.5 * proj2_b])
    p2cat = _matmul_bias(e2, w2cat, b2cat)
    C1, C2 = p2cat[:, :D], p2cat[:, D:]

    ij011, ik011 = triangle_0_1_1[0], triangle_0_1_1[1]
    ij111, ik111, kj111 = triangle_1_1_1[0], triangle_1_1_1[1], triangle_1_1_1[2]
    ij112, ik112, kj112 = triangle_1_1_2[0], triangle_1_1_2[1], triangle_1_1_2[2]
    ij122, ik122, kj122 = triangle_1_2_2[0], triangle_1_2_2[1], triangle_1_2_2[2]
    ij222, ik222, kj222 = triangle_2_2_2[0], triangle_2_2_2[1], triangle_2_2_2[2]

    acc0 = _sc_gather_scatter_small(A0, ik011.reshape(-1, 125), ij011.reshape(-1, 125), num0)

    accB = _sc_scatter_chunked([B1, C1],
                               [(0, 0, ik111, kj111, ij111),
                                (1, 1, ik122, kj122, ij122)], num1)
    a112 = _sc_scatter_chunked([B1, C1], [(0, 1, ik112, kj112, ij112)], num1)
    acc1 = accB + a112 + _sc_row_gather(a112, inverse_edge_1)

    accC = _sc_scatter_chunked([B2, C2],
                               [(0, 0, ij112, ik112, kj112),
                                (1, 1, ik222, kj222, ij222)], num2)
    a212 = _sc_scatter_chunked([B2, C2], [(0, 1, ij122, kj122, ik122)], num2)
    acc2 = accC + a212 + _sc_row_gather(a212, inverse_edge_2)

    def _bn(x, g, b):
        m = jnp.mean(x, axis=0, keepdims=True)
        v = jnp.var(x, axis=0, keepdims=True)
        return (x - m) / jnp.sqrt(v + 1e-5) * g + b

    def _head(x, W1, b1, g, bt, W2, b2, ng, nbt):
        h = _matmul_bias(x, W1, b1)
        h = jax.nn.relu(_bn(h, g, bt))
        o = _matmul_bias(h, W2, b2)
        return _bn(o, ng, nbt)

    out0 = _head((1.0 + eps0) * e0 + acc0, mlp0_W1, mlp0_b1, mlp0_g, mlp0_beta,
                 mlp0_W2, mlp0_b2, norm0_g, norm0_beta)
    out1 = _head((1.0 + eps1) * e1 + acc1, mlp1_W1, mlp1_b1, mlp1_g, mlp1_beta,
                 mlp1_W2, mlp1_b2, norm1_g, norm1_beta)
    out2 = _head((1.0 + eps2) * e2 + acc2, mlp2_W1, mlp2_b1, mlp2_g, mlp2_beta,
                 mlp2_W2, mlp2_b2, norm2_g, norm2_beta)
    return out0, out1, out2


# speedup vs baseline: 7.2223x; 7.2223x over previous
import functools

import jax
import jax.numpy as jnp
from jax import lax
from jax.experimental import pallas as pl
from jax.experimental.pallas import tpu as pltpu
from jax.experimental.pallas import tpu_sc as plsc

D = 128


def _mm_kernel(x_ref, w_ref, b_ref, o_ref):
    o_ref[...] = (jnp.dot(x_ref[...], w_ref[...], preferred_element_type=jnp.float32)
                  + b_ref[...])


def _matmul_bias(x, w, b, block=1024):
    n, d = x.shape
    k = w.shape[1]
    grid = (n + block - 1) // block
    return pl.pallas_call(
        _mm_kernel,
        grid=(grid,),
        in_specs=[
            pl.BlockSpec((block, d), lambda i: (i, 0)),
            pl.BlockSpec((d, k), lambda i: (0, 0)),
            pl.BlockSpec((1, k), lambda i: (0, 0)),
        ],
        out_specs=pl.BlockSpec((block, k), lambda i: (i, 0)),
        out_shape=jax.ShapeDtypeStruct((n, k), jnp.float32),
    )(x, w, b.reshape(1, k))


def _sc_gather_scatter_small(table, src2d, dst2d, n_out):
    nb_tile = src2d.shape[0] // 32
    gb = src2d.shape[1]
    n_pad = ((n_out + 127) // 128) * 128
    rows_tile = n_pad // 16

    mesh = plsc.VectorSubcoreMesh(core_axis_name="c", subcore_axis_name="s")

    @functools.partial(
        pl.kernel,
        out_type=jax.ShapeDtypeStruct((2, n_pad, D), jnp.float32),
        mesh=mesh,
        compiler_params=pltpu.CompilerParams(needs_layout_passes=False),
        scratch_types=[
            pltpu.VMEM_SHARED((n_pad, D), jnp.float32),
            pltpu.VMEM((nb_tile, gb), jnp.int32),
            pltpu.VMEM((nb_tile, gb), jnp.int32),
            pltpu.VMEM((gb, D), jnp.float32),
        ],
    )
    def k(table_hbm, src_hbm, dst_hbm, out_hbm, acc, sbuf, dbuf, rowbuf):
        c = lax.axis_index("c")
        s = lax.axis_index("s")
        wid = c * 16 + s
        pltpu.sync_copy(src_hbm.at[pl.ds(wid * nb_tile, nb_tile)], sbuf)
        pltpu.sync_copy(dst_hbm.at[pl.ds(wid * nb_tile, nb_tile)], dbuf)

        def _zrow(r, _):
            for l in range(D // 16):
                rowbuf[r, pl.ds(l * 16, 16)] = jnp.zeros((16,), jnp.float32)
            return 0
        lax.fori_loop(0, gb, _zrow, 0)
        for z in range(rows_tile // gb):
            pltpu.sync_copy(rowbuf, acc.at[pl.ds(s * rows_tile + z * gb, gb)])
        rem = rows_tile % gb
        if rem:
            pltpu.sync_copy(rowbuf.at[pl.ds(0, rem)],
                            acc.at[pl.ds(s * rows_tile + (rows_tile // gb) * gb, rem)])
        plsc.subcore_barrier()

        def body(j, _):
            pltpu.sync_copy(table_hbm.at[sbuf.at[j]], rowbuf)
            def _relu(r, _):
                for l in range(D // 16):
                    v = rowbuf[r, pl.ds(l * 16, 16)]
                    rowbuf[r, pl.ds(l * 16, 16)] = jnp.maximum(v, 0.0)
                return 0
            lax.fori_loop(0, gb, _relu, 0)
            pltpu.sync_copy(rowbuf, acc.at[dbuf.at[j]], add=True)
            return 0
        lax.fori_loop(0, nb_tile, body, 0)
        plsc.subcore_barrier()
        pltpu.sync_copy(acc.at[pl.ds(s * rows_tile, rows_tile)],
                        out_hbm.at[c, pl.ds(s * rows_tile, rows_tile)])

    part = k(table, src2d, dst2d)
    return part[0, :n_out] + part[1, :n_out]


_R_CHUNK = 8960
_TB = 2000
_CAP = 4224


def _sc_scatter_chunked(tables, fams, n_out):
    r = _R_CHUNK
    n_chunks = (n_out + r - 1) // r
    if n_chunks % 2:
        n_chunks += 1
    n_pad = n_chunks * r
    cap = _CAP

    mesh = plsc.VectorSubcoreMesh(core_axis_name="c", subcore_axis_name="s")

    @functools.partial(
        pl.kernel,
        out_type=jax.ShapeDtypeStruct((n_pad, D), jnp.float32),
        mesh=mesh,
        compiler_params=pltpu.CompilerParams(needs_layout_passes=False),
        scratch_types=[
            pltpu.VMEM_SHARED((r + 16, D), jnp.float32),
            pltpu.VMEM((_TB,), jnp.int32),
            pltpu.VMEM((_TB,), jnp.int32),
            pltpu.VMEM((_TB,), jnp.int32),
            pltpu.VMEM((_CAP + 16,), jnp.int32),
            pltpu.VMEM((_CAP + 16,), jnp.int32),
            pltpu.VMEM((_CAP + 16,), jnp.int32),
            pltpu.VMEM((128, D), jnp.float32),
            pltpu.VMEM((128, D), jnp.float32),
            pltpu.VMEM((128,), jnp.int32),
        ],
    )
    def k(*refs):
        n_tbl = len(tables)
        tbl = refs[:n_tbl]
        idx = refs[n_tbl:n_tbl + 3 * len(fams)]
        out_hbm = refs[n_tbl + 3 * len(fams)]
        (acc, dbat, abat, bbat, dloc, sa, sb, rowa, rowb,
         dstage) = refs[n_tbl + 3 * len(fams) + 1:]
        c = lax.axis_index("c")
        s = lax.axis_index("s")
        rows_tile = r // 16

        def _scan_fam(fi, lo):
            ia, ib, idm = idx[3 * fi], idx[3 * fi + 1], idx[3 * fi + 2]
            t16 = ia.shape[0] // 16
            base = s * t16
            trash = cap + lax.iota(jnp.int32, 16)
            rsplat = jnp.full((16,), r, jnp.int32)

            def process(cur):
                for i in range(8):
                    dloc[pl.ds(cur + i * 16, 16)] = rsplat
                    sa[pl.ds(cur + i * 16, 16)] = jnp.zeros((16,), jnp.int32)
                    sb[pl.ds(cur + i * 16, 16)] = jnp.zeros((16,), jnp.int32)
                ta, tb2 = tbl[fams[fi][0]], tbl[fams[fi][1]]

                def proc(g, _):
                    pltpu.sync_copy(ta.at[sa.at[pl.ds(g * 128, 128)]], rowa)
                    pltpu.sync_copy(tb2.at[sb.at[pl.ds(g * 128, 128)]], rowb)

                    def rr(rw, _):
                        for l in range(D // 16):
                            v = rowa[rw, pl.ds(l * 16, 16)] + rowb[rw, pl.ds(l * 16, 16)]
                            rowa[rw, pl.ds(l * 16, 16)] = jnp.maximum(v, 0.0)
                        return 0
                    lax.fori_loop(0, 128, rr, 0)
                    for i in range(8):
                        dstage[pl.ds(i * 16, 16)] = dloc[pl.ds(g * 128 + i * 16, 16)]
                    pltpu.sync_copy(rowa, acc.at[dstage], add=True)
                    return 0
                lax.fori_loop(0, (cur + 127) // 128, proc, 0)
                return jnp.int32(0)

            def batch_body(bi, cur):
                off = base + bi * _TB
                pltpu.sync_copy(idm.at[pl.ds(off, _TB)], dbat)
                pltpu.sync_copy(ia.at[pl.ds(off, _TB)], abat)
                pltpu.sync_copy(ib.at[pl.ds(off, _TB)], bbat)

                def grp(i, cur):
                    d = dbat[pl.ds(i * 16, 16)]
                    a = abat[pl.ds(i * 16, 16)]
                    b = bbat[pl.ds(i * 16, 16)]
                    m = (d >= lo) & (d < lo + r)
                    mi = m.astype(jnp.int32)
                    pos = jnp.where(m, cur + plsc.cumsum(mi) - mi, trash)
                    plsc.store_scatter(dloc, [pos], d - lo)
                    plsc.store_scatter(sa, [pos], a)
                    plsc.store_scatter(sb, [pos], b)
                    return cur + jnp.max(plsc.all_reduce_population_count(m))
                cur = lax.fori_loop(0, _TB // 16, grp, cur)
                return lax.cond(cur >= cap - _TB - 128,
                                process, lambda x: x, cur)

            cur = lax.fori_loop(0, t16 // _TB, batch_body, jnp.int32(0))
            process(cur)

        def _zidx(i, _):
            z16 = jnp.zeros((16,), jnp.int32)
            sa[pl.ds(i * 16, 16)] = z16
            sb[pl.ds(i * 16, 16)] = z16
            return 0
        lax.fori_loop(0, (_CAP + 16) // 16, _zidx, 0)

        def chunk_body(z, _):
            lo = (2 * z + c) * r
            def _zrow(rw, _):
                for l in range(D // 16):
                    rowa[rw, pl.ds(l * 16, 16)] = jnp.zeros((16,), jnp.float32)
                return 0
            lax.fori_loop(0, 128, _zrow, 0)
            nfull = rows_tile // 128
            for q in range(nfull):
                pltpu.sync_copy(rowa, acc.at[pl.ds(s * rows_tile + q * 128, 128)])
            rem = rows_tile - nfull * 128
            if rem:
                pltpu.sync_copy(rowa.at[pl.ds(0, rem)],
                                acc.at[pl.ds(s * rows_tile + nfull * 128, rem)])
            plsc.subcore_barrier()
            for fi in range(len(fams)):
                _scan_fam(fi, lo)
            plsc.subcore_barrier()
            pltpu.sync_copy(acc.at[pl.ds(s * rows_tile, rows_tile)],
                            out_hbm.at[pl.ds(lo + s * rows_tile, rows_tile)])
            return 0
        lax.fori_loop(0, n_chunks // 2, chunk_body, 0)

    args = list(tables)
    for (_, _, ia, ib, idm) in fams:
        args += [ia, ib, idm]
    return k(*args)[:n_out]


def _sc_row_gather(table, idxv):
    n = idxv.shape[0]
    n_tile = n // 32
    nfull = n_tile // 128
    tail = n_tile - nfull * 128
    cap = n_tile + 128

    mesh = plsc.VectorSubcoreMesh(core_axis_name="c", subcore_axis_name="s")

    @functools.partial(
        pl.kernel,
        out_type=jax.ShapeDtypeStruct((n, D), jnp.float32),
        mesh=mesh,
        compiler_params=pltpu.CompilerParams(needs_layout_passes=False),
        scratch_types=[
            pltpu.VMEM((cap,), jnp.int32),
            pltpu.VMEM((128, D), jnp.float32),
        ],
    )
    def k(table_hbm, idx_hbm, out_hbm, ibuf, rowbuf):
        c = lax.axis_index("c")
        s = lax.axis_index("s")
        wid = c * 16 + s
        base = wid * n_tile
        pltpu.sync_copy(idx_hbm.at[pl.ds(base, n_tile)], ibuf.at[pl.ds(0, n_tile)])
        if tail:
            for i in range(8):
                ibuf[pl.ds(n_tile + i * 16, 16)] = jnp.zeros((16,), jnp.int32)

        def body(g, _):
            pltpu.sync_copy(table_hbm.at[ibuf.at[pl.ds(g * 128, 128)]], rowbuf)
            pltpu.sync_copy(rowbuf, out_hbm.at[pl.ds(base + g * 128, 128)])
            return 0
        lax.fori_loop(0, nfull, body, 0)
        if tail:
            pltpu.sync_copy(table_hbm.at[ibuf.at[pl.ds(nfull * 128, 128)]], rowbuf)
            pltpu.sync_copy(rowbuf.at[pl.ds(0, tail)],
                            out_hbm.at[pl.ds(base + nfull * 128, tail)])

    return k(table, idxv)


def _scatter(src, idx, size):
    return jnp.zeros((size, src.shape[1]), src.dtype).at[idx].add(src)


def kernel(edge_attr0, edge_attr1, edge_attr2, edge_index0, edge_index, edge_index2,
           triangle_0_1_1, triangle_1_1_1, triangle_1_1_2, triangle_1_2_2, triangle_2_2_2,
           inverse_edge_1, inverse_edge_2,
           proj0_W, proj0_b, proj1_W, proj1_b, proj2_W, proj2_b,
           mlp0_W1, mlp0_b1, mlp0_g, mlp0_beta, mlp0_W2, mlp0_b2,
           mlp1_W1, mlp1_b1, mlp1_g, mlp1_beta, mlp1_W2, mlp1_b2,
           mlp2_W1, mlp2_b1, mlp2_g, mlp2_beta, mlp2_W2, mlp2_b2,
           norm0_g, norm0_beta, norm1_g, norm1_beta, norm2_g, norm2_beta,
           eps0, eps1, eps2):
    e0, e1, e2 = edge_attr0, edge_attr1, edge_attr2
    num0, num1, num2 = e0.shape[0], e1.shape[0], e2.shape[0]

    w1cat = jnp.concatenate([2.0 * proj0_W, proj1_W, proj2_W], axis=1)
    b1cat = jnp.concatenate([proj0_b, 0.5 * proj1_b, 0.5 * proj2_b])
    p1cat = _matmul_bias(e1, w1cat, b1cat)
    A0, B1, B2 = p1cat[:, :D], p1cat[:, D:2 * D], p1cat[:, 2 * D:]
    w2cat = jnp.concatenate([proj1_W, proj2_W], axis=1)
    b2cat = jnp.concatenate([0.5 * proj1_b, 0.5 * proj2_b])
    p2cat = _matmul_bias(e2, w2cat, b2cat)
    C1, C2 = p2cat[:, :D], p2cat[:, D:]

    ij011, ik011 = triangle_0_1_1[0], triangle_0_1_1[1]
    ij111, ik111, kj111 = triangle_1_1_1[0], triangle_1_1_1[1], triangle_1_1_1[2]
    ij112, ik112, kj112 = triangle_1_1_2[0], triangle_1_1_2[1], triangle_1_1_2[2]
    ij122, ik122, kj122 = triangle_1_2_2[0], triangle_1_2_2[1], triangle_1_2_2[2]
    ij222, ik222, kj222 = triangle_2_2_2[0], triangle_2_2_2[1], triangle_2_2_2[2]

    acc0 = _sc_gather_scatter_small(A0, ik011.reshape(-1, 125), ij011.reshape(-1, 125), num0)

    accB = _sc_scatter_chunked([B1, C1],
                               [(0, 0, ik111, kj111, ij111),
                                (1, 1, ik122, kj122, ij122)], num1)
    a112 = _sc_scatter_chunked([B1, C1], [(0, 1, ik112, kj112, ij112)], num1)
    acc1 = accB + a112 + _sc_row_gather(a112, inverse_edge_1)

    accC = _sc_scatter_chunked([B2, C2],
                               [(0, 0, ij112, ik112, kj112),
                                (1, 1, ik222, kj222, ij222)], num2)
    a212 = _sc_scatter_chunked([B2, C2], [(0, 1, ij122, kj122, ik122)], num2)
    acc2 = accC + a212 + _sc_row_gather(a212, inverse_edge_2)

    def _bn(x, g, b):
        m = jnp.mean(x, axis=0, keepdims=True)
        v = jnp.var(x, axis=0, keepdims=True)
        return (x - m) / jnp.sqrt(v + 1e-5) * g + b

    def _head(x, W1, b1, g, bt, W2, b2, ng, nbt):
        h = _matmul_bias(x, W1, b1)
        h = jax.nn.relu(_bn(h, g, bt))
        o = _matmul_bias(h, W2, b2)
        return _bn(o, ng, nbt)

    out0 = _head((1.0 + eps0) * e0 + acc0, mlp0_W1, mlp0_b1, mlp0_g, mlp0_beta,
                 mlp0_W2, mlp0_b2, norm0_g, norm0_beta)
    out1 = _head((1.0 + eps1) * e1 + acc1, mlp1_W1, mlp1_b1, mlp1_g, mlp1_beta,
                 mlp1_W2, mlp1_b2, norm1_g, norm1_beta)
    out2 = _head((1.0 + eps2) * e2 + acc2, mlp2_W1, mlp2_b1, mlp2_g, mlp2_beta,
                 mlp2_W2, mlp2_b2, norm2_g, norm2_beta)
    return out0, out1, out2


# speedup vs baseline: 9.9921x; 1.3835x over previous
import functools

import jax
import jax.numpy as jnp
from jax import lax
from jax.experimental import pallas as pl
from jax.experimental.pallas import tpu as pltpu
from jax.experimental.pallas import tpu_sc as plsc

D = 128


def _mm_kernel(x_ref, w_ref, b_ref, o_ref):
    o_ref[...] = (jnp.dot(x_ref[...], w_ref[...], preferred_element_type=jnp.float32)
                  + b_ref[...])


def _matmul_bias(x, w, b, block=1024):
    n, d = x.shape
    k = w.shape[1]
    grid = (n + block - 1) // block
    return pl.pallas_call(
        _mm_kernel,
        grid=(grid,),
        in_specs=[
            pl.BlockSpec((block, d), lambda i: (i, 0)),
            pl.BlockSpec((d, k), lambda i: (0, 0)),
            pl.BlockSpec((1, k), lambda i: (0, 0)),
        ],
        out_specs=pl.BlockSpec((block, k), lambda i: (i, 0)),
        out_shape=jax.ShapeDtypeStruct((n, k), jnp.float32),
    )(x, w, b.reshape(1, k))


def _sc_gather_scatter_small(table, src2d, dst2d, n_out):
    nb_tile = src2d.shape[0] // 32
    gb = src2d.shape[1]
    n_pad = ((n_out + 127) // 128) * 128
    rows_tile = n_pad // 16

    mesh = plsc.VectorSubcoreMesh(core_axis_name="c", subcore_axis_name="s")

    @functools.partial(
        pl.kernel,
        out_type=jax.ShapeDtypeStruct((2, n_pad, D), jnp.float32),
        mesh=mesh,
        compiler_params=pltpu.CompilerParams(needs_layout_passes=False),
        scratch_types=[
            pltpu.VMEM_SHARED((n_pad, D), jnp.float32),
            pltpu.VMEM((nb_tile, gb), jnp.int32),
            pltpu.VMEM((nb_tile, gb), jnp.int32),
            pltpu.VMEM((gb, D), jnp.float32),
        ],
    )
    def k(table_hbm, src_hbm, dst_hbm, out_hbm, acc, sbuf, dbuf, rowbuf):
        c = lax.axis_index("c")
        s = lax.axis_index("s")
        wid = c * 16 + s
        pltpu.sync_copy(src_hbm.at[pl.ds(wid * nb_tile, nb_tile)], sbuf)
        pltpu.sync_copy(dst_hbm.at[pl.ds(wid * nb_tile, nb_tile)], dbuf)

        def _zrow(r, _):
            for l in range(D // 16):
                rowbuf[r, pl.ds(l * 16, 16)] = jnp.zeros((16,), jnp.float32)
            return 0
        lax.fori_loop(0, gb, _zrow, 0)
        for z in range(rows_tile // gb):
            pltpu.sync_copy(rowbuf, acc.at[pl.ds(s * rows_tile + z * gb, gb)])
        rem = rows_tile % gb
        if rem:
            pltpu.sync_copy(rowbuf.at[pl.ds(0, rem)],
                            acc.at[pl.ds(s * rows_tile + (rows_tile // gb) * gb, rem)])
        plsc.subcore_barrier()

        def body(j, _):
            pltpu.sync_copy(table_hbm.at[sbuf.at[j]], rowbuf)
            def _relu(r, _):
                for l in range(D // 16):
                    v = rowbuf[r, pl.ds(l * 16, 16)]
                    rowbuf[r, pl.ds(l * 16, 16)] = jnp.maximum(v, 0.0)
                return 0
            lax.fori_loop(0, gb, _relu, 0)
            pltpu.sync_copy(rowbuf, acc.at[dbuf.at[j]], add=True)
            return 0
        lax.fori_loop(0, nb_tile, body, 0)
        plsc.subcore_barrier()
        pltpu.sync_copy(acc.at[pl.ds(s * rows_tile, rows_tile)],
                        out_hbm.at[c, pl.ds(s * rows_tile, rows_tile)])

    part = k(table, src2d, dst2d)
    return part[0, :n_out] + part[1, :n_out]


_R_CHUNK = 8960
_TB = 800
_CAP = 4224


def _sc_scatter_chunked(tables, fams, n_out):
    r = _R_CHUNK
    n_chunks = (n_out + r - 1) // r
    if n_chunks % 2:
        n_chunks += 1
    n_pad = n_chunks * r
    cap = _CAP

    mesh = plsc.VectorSubcoreMesh(core_axis_name="c", subcore_axis_name="s")

    @functools.partial(
        pl.kernel,
        out_type=jax.ShapeDtypeStruct((n_pad, D), jnp.float32),
        mesh=mesh,
        compiler_params=pltpu.CompilerParams(needs_layout_passes=False),
        scratch_types=[
            pltpu.VMEM_SHARED((r + 16, D), jnp.float32),
            pltpu.VMEM((_TB,), jnp.int32),
            pltpu.VMEM((_TB,), jnp.int32),
            pltpu.VMEM((_TB,), jnp.int32),
            pltpu.VMEM((_CAP + 16,), jnp.int32),
            pltpu.VMEM((_CAP + 16,), jnp.int32),
            pltpu.VMEM((_CAP + 16,), jnp.int32),
            pltpu.VMEM((64, D), jnp.float32),
            pltpu.VMEM((64, D), jnp.float32),
            pltpu.VMEM((64, D), jnp.float32),
            pltpu.VMEM((64, D), jnp.float32),
            pltpu.VMEM((64, D), jnp.float32),
            pltpu.VMEM((64,), jnp.int32),
            pltpu.SemaphoreType.DMA,
            pltpu.SemaphoreType.DMA,
            pltpu.SemaphoreType.DMA,
            pltpu.SemaphoreType.DMA,
        ],
    )
    def k(*refs):
        n_tbl = len(tables)
        tbl = refs[:n_tbl]
        idx = refs[n_tbl:n_tbl + 3 * len(fams)]
        out_hbm = refs[n_tbl + 3 * len(fams)]
        (acc, dbat, abat, bbat, dloc, sa, sb, rowz, rowa, rowb, rowc, rowd,
         dstage, sem0, sem1, sem2, sem3) = refs[n_tbl + 3 * len(fams) + 1:]
        c = lax.axis_index("c")
        s = lax.axis_index("s")
        rows_tile = r // 16

        def _scan_fam(fi, lo):
            ia, ib, idm = idx[3 * fi], idx[3 * fi + 1], idx[3 * fi + 2]
            t16 = ia.shape[0] // 16
            base = s * t16
            trash = cap + lax.iota(jnp.int32, 16)
            rsplat = jnp.full((16,), r, jnp.int32)

            def process(cur):
                for i in range(4):
                    dloc[pl.ds(cur + i * 16, 16)] = rsplat
                    sa[pl.ds(cur + i * 16, 16)] = jnp.zeros((16,), jnp.int32)
                    sb[pl.ds(cur + i * 16, 16)] = jnp.zeros((16,), jnp.int32)
                ng = (cur + 63) // 64
                ta, tb2 = tbl[fams[fi][0]], tbl[fams[fi][1]]

                @pl.when(ng > 0)
                def _():
                    pltpu.async_copy(ta.at[sa.at[pl.ds(0, 64)]], rowa, sem0)
                    pltpu.async_copy(tb2.at[sb.at[pl.ds(0, 64)]], rowb, sem1)

                def stream(g, rA, rB, sA, sB, rA2, rB2, sA2, sB2):
                    pltpu.make_async_copy(ta.at[sa.at[pl.ds(g * 64, 64)]], rA, sA).wait()
                    pltpu.make_async_copy(tb2.at[sb.at[pl.ds(g * 64, 64)]], rB, sB).wait()

                    @pl.when(g + 1 < ng)
                    def _():
                        pltpu.async_copy(ta.at[sa.at[pl.ds((g + 1) * 64, 64)]], rA2, sA2)
                        pltpu.async_copy(tb2.at[sb.at[pl.ds((g + 1) * 64, 64)]], rB2, sB2)

                    def rr(rw, _):
                        for l in range(D // 16):
                            v = rA[rw, pl.ds(l * 16, 16)] + rB[rw, pl.ds(l * 16, 16)]
                            rA[rw, pl.ds(l * 16, 16)] = jnp.maximum(v, 0.0)
                        return 0
                    lax.fori_loop(0, 64, rr, 0)
                    for i in range(4):
                        dstage[pl.ds(i * 16, 16)] = dloc[pl.ds(g * 64 + i * 16, 16)]
                    pltpu.sync_copy(rA, acc.at[dstage], add=True)
                    return 0

                def proc(g, _):
                    return lax.cond(
                        g % 2 == 0,
                        lambda: stream(g, rowa, rowb, sem0, sem1, rowc, rowd, sem2, sem3),
                        lambda: stream(g, rowc, rowd, sem2, sem3, rowa, rowb, sem0, sem1))
                lax.fori_loop(0, ng, proc, 0)
                return jnp.int32(0)

            def batch_body(bi, cur):
                off = base + bi * _TB
                cpd = pltpu.async_copy(idm.at[pl.ds(off, _TB)], dbat, sem0)
                cpa = pltpu.async_copy(ia.at[pl.ds(off, _TB)], abat, sem1)
                cpb = pltpu.async_copy(ib.at[pl.ds(off, _TB)], bbat, sem2)
                cpd.wait()
                cpa.wait()
                cpb.wait()

                def grp(i, cur):
                    d = dbat[pl.ds(i * 16, 16)]
                    a = abat[pl.ds(i * 16, 16)]
                    b = bbat[pl.ds(i * 16, 16)]
                    m = (d >= lo) & (d < lo + r)
                    mi = m.astype(jnp.int32)
                    pos = jnp.where(m, cur + plsc.cumsum(mi) - mi, trash)
                    plsc.store_scatter(dloc, [pos], d - lo)
                    plsc.store_scatter(sa, [pos], a)
                    plsc.store_scatter(sb, [pos], b)
                    return cur + jnp.max(plsc.all_reduce_population_count(m))
                cur = lax.fori_loop(0, _TB // 16, grp, cur)
                return lax.cond(cur >= cap - _TB - 128,
                                process, lambda x: x, cur)

            cur = lax.fori_loop(0, t16 // _TB, batch_body, jnp.int32(0))
            process(cur)

        def _zidx(i, _):
            z16 = jnp.zeros((16,), jnp.int32)
            sa[pl.ds(i * 16, 16)] = z16
            sb[pl.ds(i * 16, 16)] = z16
            return 0
        lax.fori_loop(0, (_CAP + 16) // 16, _zidx, 0)

        def chunk_body(z, _):
            lo = (2 * z + c) * r
            def _zrow(rw, _):
                for l in range(D // 16):
                    rowz[rw, pl.ds(l * 16, 16)] = jnp.zeros((16,), jnp.float32)
                return 0
            lax.fori_loop(0, 64, _zrow, 0)
            nfull = rows_tile // 64
            for q in range(nfull):
                pltpu.sync_copy(rowz, acc.at[pl.ds(s * rows_tile + q * 64, 64)])
            rem = rows_tile - nfull * 64
            if rem:
                pltpu.sync_copy(rowz.at[pl.ds(0, rem)],
                                acc.at[pl.ds(s * rows_tile + nfull * 64, rem)])
            plsc.subcore_barrier()
            for fi in range(len(fams)):
                _scan_fam(fi, lo)
            plsc.subcore_barrier()
            pltpu.sync_copy(acc.at[pl.ds(s * rows_tile, rows_tile)],
                            out_hbm.at[pl.ds(lo + s * rows_tile, rows_tile)])
            return 0
        lax.fori_loop(0, n_chunks // 2, chunk_body, 0)

    args = list(tables)
    for (_, _, ia, ib, idm) in fams:
        args += [ia, ib, idm]
    return k(*args)[:n_out]


def _sc_row_gather(table, idxv):
    n = idxv.shape[0]
    n_tile = n // 32
    nfull = n_tile // 128
    tail = n_tile - nfull * 128
    cap = n_tile + 128

    mesh = plsc.VectorSubcoreMesh(core_axis_name="c", subcore_axis_name="s")

    @functools.partial(
        pl.kernel,
        out_type=jax.ShapeDtypeStruct((n, D), jnp.float32),
        mesh=mesh,
        compiler_params=pltpu.CompilerParams(needs_layout_passes=False),
        scratch_types=[
            pltpu.VMEM((cap,), jnp.int32),
            pltpu.VMEM((128, D), jnp.float32),
        ],
    )
    def k(table_hbm, idx_hbm, out_hbm, ibuf, rowbuf):
        c = lax.axis_index("c")
        s = lax.axis_index("s")
        wid = c * 16 + s
        base = wid * n_tile
        pltpu.sync_copy(idx_hbm.at[pl.ds(base, n_tile)], ibuf.at[pl.ds(0, n_tile)])
        if tail:
            for i in range(8):
                ibuf[pl.ds(n_tile + i * 16, 16)] = jnp.zeros((16,), jnp.int32)

        def body(g, _):
            pltpu.sync_copy(table_hbm.at[ibuf.at[pl.ds(g * 128, 128)]], rowbuf)
            pltpu.sync_copy(rowbuf, out_hbm.at[pl.ds(base + g * 128, 128)])
            return 0
        lax.fori_loop(0, nfull, body, 0)
        if tail:
            pltpu.sync_copy(table_hbm.at[ibuf.at[pl.ds(nfull * 128, 128)]], rowbuf)
            pltpu.sync_copy(rowbuf.at[pl.ds(0, tail)],
                            out_hbm.at[pl.ds(base + nfull * 128, tail)])

    return k(table, idxv)


def _scatter(src, idx, size):
    return jnp.zeros((size, src.shape[1]), src.dtype).at[idx].add(src)


def kernel(edge_attr0, edge_attr1, edge_attr2, edge_index0, edge_index, edge_index2,
           triangle_0_1_1, triangle_1_1_1, triangle_1_1_2, triangle_1_2_2, triangle_2_2_2,
           inverse_edge_1, inverse_edge_2,
           proj0_W, proj0_b, proj1_W, proj1_b, proj2_W, proj2_b,
           mlp0_W1, mlp0_b1, mlp0_g, mlp0_beta, mlp0_W2, mlp0_b2,
           mlp1_W1, mlp1_b1, mlp1_g, mlp1_beta, mlp1_W2, mlp1_b2,
           mlp2_W1, mlp2_b1, mlp2_g, mlp2_beta, mlp2_W2, mlp2_b2,
           norm0_g, norm0_beta, norm1_g, norm1_beta, norm2_g, norm2_beta,
           eps0, eps1, eps2):
    e0, e1, e2 = edge_attr0, edge_attr1, edge_attr2
    num0, num1, num2 = e0.shape[0], e1.shape[0], e2.shape[0]

    w1cat = jnp.concatenate([2.0 * proj0_W, proj1_W, proj2_W], axis=1)
    b1cat = jnp.concatenate([proj0_b, 0.5 * proj1_b, 0.5 * proj2_b])
    p1cat = _matmul_bias(e1, w1cat, b1cat)
    A0, B1, B2 = p1cat[:, :D], p1cat[:, D:2 * D], p1cat[:, 2 * D:]
    w2cat = jnp.concatenate([proj1_W, proj2_W], axis=1)
    b2cat = jnp.concatenate([0.5 * proj1_b, 0.5 * proj2_b])
    p2cat = _matmul_bias(e2, w2cat, b2cat)
    C1, C2 = p2cat[:, :D], p2cat[:, D:]

    ij011, ik011 = triangle_0_1_1[0], triangle_0_1_1[1]
    ij111, ik111, kj111 = triangle_1_1_1[0], triangle_1_1_1[1], triangle_1_1_1[2]
    ij112, ik112, kj112 = triangle_1_1_2[0], triangle_1_1_2[1], triangle_1_1_2[2]
    ij122, ik122, kj122 = triangle_1_2_2[0], triangle_1_2_2[1], triangle_1_2_2[2]
    ij222, ik222, kj222 = triangle_2_2_2[0], triangle_2_2_2[1], triangle_2_2_2[2]

    acc0 = _sc_gather_scatter_small(A0, ik011.reshape(-1, 125), ij011.reshape(-1, 125), num0)

    accB = _sc_scatter_chunked([B1, C1],
                               [(0, 0, ik111, kj111, ij111),
                                (1, 1, ik122, kj122, ij122)], num1)
    a112 = _sc_scatter_chunked([B1, C1], [(0, 1, ik112, kj112, ij112)], num1)
    acc1 = accB + a112 + _sc_row_gather(a112, inverse_edge_1)

    accC = _sc_scatter_chunked([B2, C2],
                               [(0, 0, ij112, ik112, kj112),
                                (1, 1, ik222, kj222, ij222)], num2)
    a212 = _sc_scatter_chunked([B2, C2], [(0, 1, ij122, kj122, ik122)], num2)
    acc2 = accC + a212 + _sc_row_gather(a212, inverse_edge_2)

    def _bn(x, g, b):
        m = jnp.mean(x, axis=0, keepdims=True)
        v = jnp.var(x, axis=0, keepdims=True)
        return (x - m) / jnp.sqrt(v + 1e-5) * g + b

    def _head(x, W1, b1, g, bt, W2, b2, ng, nbt):
        h = _matmul_bias(x, W1, b1)
        h = jax.nn.relu(_bn(h, g, bt))
        o = _matmul_bias(h, W2, b2)
        return _bn(o, ng, nbt)

    out0 = _head((1.0 + eps0) * e0 + acc0, mlp0_W1, mlp0_b1, mlp0_g, mlp0_beta,
                 mlp0_W2, mlp0_b2, norm0_g, norm0_beta)
    out1 = _head((1.0 + eps1) * e1 + acc1, mlp1_W1, mlp1_b1, mlp1_g, mlp1_beta,
                 mlp1_W2, mlp1_b2, norm1_g, norm1_beta)
    out2 = _head((1.0 + eps2) * e2 + acc2, mlp2_W1, mlp2_b1, mlp2_g, mlp2_beta,
                 mlp2_W2, mlp2_b2, norm2_g, norm2_beta)
    return out0, out1, out2


# speedup vs baseline: 10.2915x; 1.0300x over previous
import functools

import jax
import jax.numpy as jnp
from jax import lax
from jax.experimental import pallas as pl
from jax.experimental.pallas import tpu as pltpu
from jax.experimental.pallas import tpu_sc as plsc

D = 128


def _mm_kernel(x_ref, w_ref, b_ref, o_ref):
    o_ref[...] = (jnp.dot(x_ref[...], w_ref[...], preferred_element_type=jnp.float32)
                  + b_ref[...])


def _mm_multi_kernel(x_ref, w_ref, b_ref, *o_refs):
    o = (jnp.dot(x_ref[...], w_ref[...], preferred_element_type=jnp.float32)
         + b_ref[...])
    for j, o_ref in enumerate(o_refs):
        o_ref[...] = o[:, j * D:(j + 1) * D]


def _matmul_bias_multi(x, w, b, n_out, block=1024):
    n, d = x.shape
    k = w.shape[1]
    grid = (n + block - 1) // block
    return pl.pallas_call(
        functools.partial(_mm_multi_kernel),
        grid=(grid,),
        in_specs=[
            pl.BlockSpec((block, d), lambda i: (i, 0)),
            pl.BlockSpec((d, k), lambda i: (0, 0)),
            pl.BlockSpec((1, k), lambda i: (0, 0)),
        ],
        out_specs=[pl.BlockSpec((block, D), lambda i: (i, 0))] * n_out,
        out_shape=[jax.ShapeDtypeStruct((n, D), jnp.float32)] * n_out,
    )(x, w, b.reshape(1, k))


def _matmul_bias(x, w, b, block=1024):
    n, d = x.shape
    k = w.shape[1]
    grid = (n + block - 1) // block
    return pl.pallas_call(
        _mm_kernel,
        grid=(grid,),
        in_specs=[
            pl.BlockSpec((block, d), lambda i: (i, 0)),
            pl.BlockSpec((d, k), lambda i: (0, 0)),
            pl.BlockSpec((1, k), lambda i: (0, 0)),
        ],
        out_specs=pl.BlockSpec((block, k), lambda i: (i, 0)),
        out_shape=jax.ShapeDtypeStruct((n, k), jnp.float32),
    )(x, w, b.reshape(1, k))


def _sc_gather_scatter_small(table, src2d, dst2d, n_out):
    nb_tile = src2d.shape[0] // 32
    gb = src2d.shape[1]
    n_pad = ((n_out + 127) // 128) * 128
    rows_tile = n_pad // 16

    mesh = plsc.VectorSubcoreMesh(core_axis_name="c", subcore_axis_name="s")

    @functools.partial(
        pl.kernel,
        out_type=jax.ShapeDtypeStruct((2, n_pad, D), jnp.float32),
        mesh=mesh,
        compiler_params=pltpu.CompilerParams(needs_layout_passes=False),
        scratch_types=[
            pltpu.VMEM_SHARED((n_pad, D), jnp.float32),
            pltpu.VMEM((nb_tile, gb), jnp.int32),
            pltpu.VMEM((nb_tile, gb), jnp.int32),
            pltpu.VMEM((gb, D), jnp.float32),
        ],
    )
    def k(table_hbm, src_hbm, dst_hbm, out_hbm, acc, sbuf, dbuf, rowbuf):
        c = lax.axis_index("c")
        s = lax.axis_index("s")
        wid = c * 16 + s
        pltpu.sync_copy(src_hbm.at[pl.ds(wid * nb_tile, nb_tile)], sbuf)
        pltpu.sync_copy(dst_hbm.at[pl.ds(wid * nb_tile, nb_tile)], dbuf)

        def _zrow(r, _):
            for l in range(D // 16):
                rowbuf[r, pl.ds(l * 16, 16)] = jnp.zeros((16,), jnp.float32)
            return 0
        lax.fori_loop(0, gb, _zrow, 0)
        for z in range(rows_tile // gb):
            pltpu.sync_copy(rowbuf, acc.at[pl.ds(s * rows_tile + z * gb, gb)])
        rem = rows_tile % gb
        if rem:
            pltpu.sync_copy(rowbuf.at[pl.ds(0, rem)],
                            acc.at[pl.ds(s * rows_tile + (rows_tile // gb) * gb, rem)])
        plsc.subcore_barrier()

        def body(j, _):
            pltpu.sync_copy(table_hbm.at[sbuf.at[j]], rowbuf)
            def _relu(r, _):
                for l in range(D // 16):
                    v = rowbuf[r, pl.ds(l * 16, 16)]
                    rowbuf[r, pl.ds(l * 16, 16)] = jnp.maximum(v, 0.0)
                return 0
            lax.fori_loop(0, gb, _relu, 0)
            pltpu.sync_copy(rowbuf, acc.at[dbuf.at[j]], add=True)
            return 0
        lax.fori_loop(0, nb_tile, body, 0)
        plsc.subcore_barrier()
        pltpu.sync_copy(acc.at[pl.ds(s * rows_tile, rows_tile)],
                        out_hbm.at[c, pl.ds(s * rows_tile, rows_tile)])

    part = k(table, src2d, dst2d)
    return part[0, :n_out] + part[1, :n_out]


_R_CHUNK = 8960
_TB = 800
_CAP = 4224


def _sc_scatter_chunked(tables, fams, n_out):
    r = _R_CHUNK
    n_chunks = (n_out + r - 1) // r
    if n_chunks % 2:
        n_chunks += 1
    n_pad = n_chunks * r
    cap = _CAP

    mesh = plsc.VectorSubcoreMesh(core_axis_name="c", subcore_axis_name="s")

    @functools.partial(
        pl.kernel,
        out_type=jax.ShapeDtypeStruct((n_pad, D), jnp.float32),
        mesh=mesh,
        compiler_params=pltpu.CompilerParams(needs_layout_passes=False),
        scratch_types=[
            pltpu.VMEM_SHARED((r + 16, D), jnp.float32),
            pltpu.VMEM((_TB,), jnp.int32),
            pltpu.VMEM((_TB,), jnp.int32),
            pltpu.VMEM((_TB,), jnp.int32),
            pltpu.VMEM((_CAP + 16,), jnp.int32),
            pltpu.VMEM((_CAP + 16,), jnp.int32),
            pltpu.VMEM((_CAP + 16,), jnp.int32),
            pltpu.VMEM((64, D), jnp.float32),
            pltpu.VMEM((64, D), jnp.float32),
            pltpu.VMEM((64, D), jnp.float32),
            pltpu.VMEM((64, D), jnp.float32),
            pltpu.VMEM((64, D), jnp.float32),
            pltpu.VMEM((64,), jnp.int32),
            pltpu.SemaphoreType.DMA,
            pltpu.SemaphoreType.DMA,
            pltpu.SemaphoreType.DMA,
            pltpu.SemaphoreType.DMA,
        ],
    )
    def k(*refs):
        n_tbl = len(tables)
        tbl = refs[:n_tbl]
        idx = refs[n_tbl:n_tbl + 3 * len(fams)]
        out_hbm = refs[n_tbl + 3 * len(fams)]
        (acc, dbat, abat, bbat, dloc, sa, sb, rowz, rowa, rowb, rowc, rowd,
         dstage, sem0, sem1, sem2, sem3) = refs[n_tbl + 3 * len(fams) + 1:]
        c = lax.axis_index("c")
        s = lax.axis_index("s")
        rows_tile = r // 16

        def _scan_fam(fi, lo):
            ia, ib, idm = idx[3 * fi], idx[3 * fi + 1], idx[3 * fi + 2]
            t16 = ia.shape[0] // 16
            base = s * t16
            trash = cap + lax.iota(jnp.int32, 16)
            rsplat = jnp.full((16,), r, jnp.int32)

            def process(cur):
                for i in range(4):
                    dloc[pl.ds(cur + i * 16, 16)] = rsplat
                    sa[pl.ds(cur + i * 16, 16)] = jnp.zeros((16,), jnp.int32)
                    sb[pl.ds(cur + i * 16, 16)] = jnp.zeros((16,), jnp.int32)
                ng = (cur + 63) // 64
                ta, tb2 = tbl[fams[fi][0]], tbl[fams[fi][1]]

                @pl.when(ng > 0)
                def _():
                    pltpu.async_copy(ta.at[sa.at[pl.ds(0, 64)]], rowa, sem0)
                    pltpu.async_copy(tb2.at[sb.at[pl.ds(0, 64)]], rowb, sem1)

                def stream(g, rA, rB, sA, sB, rA2, rB2, sA2, sB2):
                    pltpu.make_async_copy(ta.at[sa.at[pl.ds(g * 64, 64)]], rA, sA).wait()
                    pltpu.make_async_copy(tb2.at[sb.at[pl.ds(g * 64, 64)]], rB, sB).wait()

                    @pl.when(g + 1 < ng)
                    def _():
                        pltpu.async_copy(ta.at[sa.at[pl.ds((g + 1) * 64, 64)]], rA2, sA2)
                        pltpu.async_copy(tb2.at[sb.at[pl.ds((g + 1) * 64, 64)]], rB2, sB2)

                    def rr(rw, _):
                        for l in range(D // 16):
                            v = rA[rw, pl.ds(l * 16, 16)] + rB[rw, pl.ds(l * 16, 16)]
                            rA[rw, pl.ds(l * 16, 16)] = jnp.maximum(v, 0.0)
                        return 0
                    lax.fori_loop(0, 64, rr, 0)
                    for i in range(4):
                        dstage[pl.ds(i * 16, 16)] = dloc[pl.ds(g * 64 + i * 16, 16)]
                    pltpu.sync_copy(rA, acc.at[dstage], add=True)
                    return 0

                def proc(g, _):
                    return lax.cond(
                        g % 2 == 0,
                        lambda: stream(g, rowa, rowb, sem0, sem1, rowc, rowd, sem2, sem3),
                        lambda: stream(g, rowc, rowd, sem2, sem3, rowa, rowb, sem0, sem1))
                lax.fori_loop(0, ng, proc, 0)
                return jnp.int32(0)

            def batch_body(bi, cur):
                off = base + bi * _TB
                cpd = pltpu.async_copy(idm.at[pl.ds(off, _TB)], dbat, sem0)
                cpa = pltpu.async_copy(ia.at[pl.ds(off, _TB)], abat, sem1)
                cpb = pltpu.async_copy(ib.at[pl.ds(off, _TB)], bbat, sem2)
                cpd.wait()
                cpa.wait()
                cpb.wait()

                def grp(i, cur):
                    d = dbat[pl.ds(i * 16, 16)]
                    a = abat[pl.ds(i * 16, 16)]
                    b = bbat[pl.ds(i * 16, 16)]
                    m = (d >= lo) & (d < lo + r)
                    mi = m.astype(jnp.int32)
                    pos = jnp.where(m, cur + plsc.cumsum(mi) - mi, trash)
                    plsc.store_scatter(dloc, [pos], d - lo)
                    plsc.store_scatter(sa, [pos], a)
                    plsc.store_scatter(sb, [pos], b)
                    return cur + jnp.max(plsc.all_reduce_population_count(m))
                cur = lax.fori_loop(0, _TB // 16, grp, cur)
                return lax.cond(cur >= cap - _TB - 128,
                                process, lambda x: x, cur)

            cur = lax.fori_loop(0, t16 // _TB, batch_body, jnp.int32(0))
            process(cur)

        def _zidx(i, _):
            z16 = jnp.zeros((16,), jnp.int32)
            sa[pl.ds(i * 16, 16)] = z16
            sb[pl.ds(i * 16, 16)] = z16
            return 0
        lax.fori_loop(0, (_CAP + 16) // 16, _zidx, 0)

        def chunk_body(z, _):
            lo = (2 * z + c) * r
            def _zrow(rw, _):
                for l in range(D // 16):
                    rowz[rw, pl.ds(l * 16, 16)] = jnp.zeros((16,), jnp.float32)
                return 0
            lax.fori_loop(0, 64, _zrow, 0)
            nfull = rows_tile // 64
            for q in range(nfull):
                pltpu.sync_copy(rowz, acc.at[pl.ds(s * rows_tile + q * 64, 64)])
            rem = rows_tile - nfull * 64
            if rem:
                pltpu.sync_copy(rowz.at[pl.ds(0, rem)],
                                acc.at[pl.ds(s * rows_tile + nfull * 64, rem)])
            plsc.subcore_barrier()
            for fi in range(len(fams)):
                _scan_fam(fi, lo)
            plsc.subcore_barrier()
            pltpu.sync_copy(acc.at[pl.ds(s * rows_tile, rows_tile)],
                            out_hbm.at[pl.ds(lo + s * rows_tile, rows_tile)])
            return 0
        lax.fori_loop(0, n_chunks // 2, chunk_body, 0)

    args = list(tables)
    for (_, _, ia, ib, idm) in fams:
        args += [ia, ib, idm]
    return k(*args)[:n_out]


def _sc_row_gather(table, idxv):
    n = idxv.shape[0]
    n_tile = n // 32
    nfull = n_tile // 128
    tail = n_tile - nfull * 128
    cap = n_tile + 128

    mesh = plsc.VectorSubcoreMesh(core_axis_name="c", subcore_axis_name="s")

    @functools.partial(
        pl.kernel,
        out_type=jax.ShapeDtypeStruct((n, D), jnp.float32),
        mesh=mesh,
        compiler_params=pltpu.CompilerParams(needs_layout_passes=False),
        scratch_types=[
            pltpu.VMEM((cap,), jnp.int32),
            pltpu.VMEM((128, D), jnp.float32),
        ],
    )
    def k(table_hbm, idx_hbm, out_hbm, ibuf, rowbuf):
        c = lax.axis_index("c")
        s = lax.axis_index("s")
        wid = c * 16 + s
        base = wid * n_tile
        pltpu.sync_copy(idx_hbm.at[pl.ds(base, n_tile)], ibuf.at[pl.ds(0, n_tile)])
        if tail:
            for i in range(8):
                ibuf[pl.ds(n_tile + i * 16, 16)] = jnp.zeros((16,), jnp.int32)

        def body(g, _):
            pltpu.sync_copy(table_hbm.at[ibuf.at[pl.ds(g * 128, 128)]], rowbuf)
            pltpu.sync_copy(rowbuf, out_hbm.at[pl.ds(base + g * 128, 128)])
            return 0
        lax.fori_loop(0, nfull, body, 0)
        if tail:
            pltpu.sync_copy(table_hbm.at[ibuf.at[pl.ds(nfull * 128, 128)]], rowbuf)
            pltpu.sync_copy(rowbuf.at[pl.ds(0, tail)],
                            out_hbm.at[pl.ds(base + nfull * 128, tail)])

    return k(table, idxv)


def _scatter(src, idx, size):
    return jnp.zeros((size, src.shape[1]), src.dtype).at[idx].add(src)


def kernel(edge_attr0, edge_attr1, edge_attr2, edge_index0, edge_index, edge_index2,
           triangle_0_1_1, triangle_1_1_1, triangle_1_1_2, triangle_1_2_2, triangle_2_2_2,
           inverse_edge_1, inverse_edge_2,
           proj0_W, proj0_b, proj1_W, proj1_b, proj2_W, proj2_b,
           mlp0_W1, mlp0_b1, mlp0_g, mlp0_beta, mlp0_W2, mlp0_b2,
           mlp1_W1, mlp1_b1, mlp1_g, mlp1_beta, mlp1_W2, mlp1_b2,
           mlp2_W1, mlp2_b1, mlp2_g, mlp2_beta, mlp2_W2, mlp2_b2,
           norm0_g, norm0_beta, norm1_g, norm1_beta, norm2_g, norm2_beta,
           eps0, eps1, eps2):
    e0, e1, e2 = edge_attr0, edge_attr1, edge_attr2
    num0, num1, num2 = e0.shape[0], e1.shape[0], e2.shape[0]

    w1cat = jnp.concatenate([2.0 * proj0_W, proj1_W, proj2_W], axis=1)
    b1cat = jnp.concatenate([proj0_b, 0.5 * proj1_b, 0.5 * proj2_b])
    A0, B1, B2 = _matmul_bias_multi(e1, w1cat, b1cat, 3)
    w2cat = jnp.concatenate([proj1_W, proj2_W], axis=1)
    b2cat = jnp.concatenate([0.5 * proj1_b, 0.5 * proj2_b])
    C1, C2 = _matmul_bias_multi(e2, w2cat, b2cat, 2)

    ij011, ik011 = triangle_0_1_1[0], triangle_0_1_1[1]
    ij111, ik111, kj111 = triangle_1_1_1[0], triangle_1_1_1[1], triangle_1_1_1[2]
    ij112, ik112, kj112 = triangle_1_1_2[0], triangle_1_1_2[1], triangle_1_1_2[2]
    ij122, ik122, kj122 = triangle_1_2_2[0], triangle_1_2_2[1], triangle_1_2_2[2]
    ij222, ik222, kj222 = triangle_2_2_2[0], triangle_2_2_2[1], triangle_2_2_2[2]

    acc0 = _sc_gather_scatter_small(A0, ik011.reshape(-1, 125), ij011.reshape(-1, 125), num0)

    accB = _sc_scatter_chunked([B1, C1],
                               [(0, 0, ik111, kj111, ij111),
                                (1, 1, ik122, kj122, ij122)], num1)
    a112 = _sc_scatter_chunked([B1, C1], [(0, 1, ik112, kj112, ij112)], num1)
    acc1 = accB + a112 + _sc_row_gather(a112, inverse_edge_1)

    accC = _sc_scatter_chunked([B2, C2],
                               [(0, 0, ij112, ik112, kj112),
                                (1, 1, ik222, kj222, ij222)], num2)
    a212 = _sc_scatter_chunked([B2, C2], [(0, 1, ij122, kj122, ik122)], num2)
    acc2 = accC + a212 + _sc_row_gather(a212, inverse_edge_2)

    def _bn(x, g, b):
        m = jnp.mean(x, axis=0, keepdims=True)
        v = jnp.var(x, axis=0, keepdims=True)
        return (x - m) / jnp.sqrt(v + 1e-5) * g + b

    def _head(x, W1, b1, g, bt, W2, b2, ng, nbt):
        h = _matmul_bias(x, W1, b1)
        h = jax.nn.relu(_bn(h, g, bt))
        o = _matmul_bias(h, W2, b2)
        return _bn(o, ng, nbt)

    out0 = _head((1.0 + eps0) * e0 + acc0, mlp0_W1, mlp0_b1, mlp0_g, mlp0_beta,
                 mlp0_W2, mlp0_b2, norm0_g, norm0_beta)
    out1 = _head((1.0 + eps1) * e1 + acc1, mlp1_W1, mlp1_b1, mlp1_g, mlp1_beta,
                 mlp1_W2, mlp1_b2, norm1_g, norm1_beta)
    out2 = _head((1.0 + eps2) * e2 + acc2, mlp2_W1, mlp2_b1, mlp2_g, mlp2_beta,
                 mlp2_W2, mlp2_b2, norm2_g, norm2_beta)
    return out0, out1, out2
